# Initial kernel scaffold; baseline (speedup 1.0000x reference)
import jax
import jax.numpy as jnp
from jax.experimental import pallas as pl


def kernel(x, edge_index, W1, att_src1, att_dst1, b1, W2, att_src2, att_dst2, b2):
    raise NotImplementedError("write your pallas kernel here")


# speedup vs baseline: 1.0157x; 1.0157x over previous
import functools

import jax
import jax.numpy as jnp
import numpy as np
from jax.experimental import pallas as pl
from jax.experimental.pallas import tpu as pltpu

N = 10000
E = 320000
IN_DIM = 128
HID = 16
HEADS = 8
OUT_DIM = 128

ROW_BLK = 2000


def _mm_body(x_ref, w_ref, o_ref):
    o_ref[...] = jnp.dot(x_ref[...], w_ref[...], preferred_element_type=jnp.float32)


def _mm(x, w):
    n, k = x.shape
    m = w.shape[1]
    return pl.pallas_call(
        _mm_body,
        grid=(n // ROW_BLK,),
        in_specs=[
            pl.BlockSpec((ROW_BLK, k), lambda i: (i, 0)),
            pl.BlockSpec((k, m), lambda i: (0, 0)),
        ],
        out_specs=pl.BlockSpec((ROW_BLK, m), lambda i: (i, 0)),
        out_shape=jax.ShapeDtypeStruct((n, m), jnp.float32),
    )(x, w)


def _gat_layer(x, src, dst, W, att_src, att_dst, bias, heads, out_ch, concat):
    n = x.shape[0]
    h = _mm(x, W).reshape(n, heads, out_ch)
    a_src = (h * att_src[None]).sum(-1)
    a_dst = (h * att_dst[None]).sum(-1)
    alpha = a_src[src] + a_dst[dst]
    alpha = jnp.where(alpha >= 0, alpha, 0.2 * alpha)
    amax = jax.ops.segment_max(alpha, dst, num_segments=n)
    amax = jnp.where(jnp.isfinite(amax), amax, 0.0)
    ex = jnp.exp(alpha - amax[dst])
    denom = jax.ops.segment_sum(ex, dst, num_segments=n)
    coef = ex / (denom[dst] + 1e-16)
    msg = h[src] * coef[..., None]
    out = jax.ops.segment_sum(msg, dst, num_segments=n)
    if concat:
        out = out.reshape(n, heads * out_ch)
    else:
        out = out.mean(axis=1)
    return out + bias


def kernel(x, edge_index, W1, att_src1, att_dst1, b1, W2, att_src2, att_dst2, b2):
    src = edge_index[0]
    dst = edge_index[1]
    h = _gat_layer(x, src, dst, W1, att_src1, att_dst1, b1, HEADS, HID, True)
    h = jax.nn.elu(h)
    out = _gat_layer(h, src, dst, W2, att_src2, att_dst2, b2, 1, OUT_DIM, False)
    return out


# speedup vs baseline: 1.0158x; 1.0001x over previous
import jax
import jax.numpy as jnp
from jax.experimental import pallas as pl

N = 10000
E = 320000
IN_DIM = 128
HID = 16
HEADS = 8
OUT_DIM = 128

ROW_BLK = 2000


def _mm_body(x_ref, w_ref, o_ref):
    o_ref[...] = jnp.dot(x_ref[...], w_ref[...],
                         preferred_element_type=jnp.float32)


def _mm(x, w):
    n, k = x.shape
    m = w.shape[1]
    return pl.pallas_call(
        _mm_body,
        grid=(n // ROW_BLK,),
        in_specs=[
            pl.BlockSpec((ROW_BLK, k), lambda i: (i, 0)),
            pl.BlockSpec((k, m), lambda i: (0, 0)),
        ],
        out_specs=pl.BlockSpec((ROW_BLK, m), lambda i: (i, 0)),
        out_shape=jax.ShapeDtypeStruct((n, m), jnp.float32),
    )(x, w)


def _gat_layer(x, src, dst, W, att_src, att_dst, bias, heads, out_ch, concat):
    n = x.shape[0]
    h = _mm(x, W).reshape(n, heads, out_ch)
    a_src = (h * att_src[None]).sum(-1)
    a_dst = (h * att_dst[None]).sum(-1)
    alpha = a_src[src] + a_dst[dst]
    alpha = jnp.where(alpha >= 0, alpha, 0.2 * alpha)
    amax = jax.ops.segment_max(alpha, dst, num_segments=n)
    amax = jnp.where(jnp.isfinite(amax), amax, 0.0)
    ex = jnp.exp(alpha - amax[dst])
    denom = jax.ops.segment_sum(ex, dst, num_segments=n)
    coef = ex / (denom[dst] + 1e-16)
    msg = h[src] * coef[..., None]
    out = jax.ops.segment_sum(msg, dst, num_segments=n)
    if concat:
        out = out.reshape(n, heads * out_ch)
    else:
        out = out.mean(axis=1)
    return out + bias


def kernel(x, edge_index, W1, att_src1, att_dst1, b1, W2, att_src2, att_dst2, b2):
    src = edge_index[0]
    dst = edge_index[1]
    h = _gat_layer(x, src, dst, W1, att_src1, att_dst1, b1, HEADS, HID, True)
    h = jax.nn.elu(h)
    out = _gat_layer(h, src, dst, W2, att_src2, att_dst2, b2, 1, OUT_DIM, False)
    return out
